# baseline (speedup 1.0000x reference)
import functools

import jax
import jax.numpy as jnp
from jax import lax
from jax.experimental import pallas as pl
from jax.experimental.pallas import tpu as pltpu
from jax.experimental.pallas import tpu_sc as plsc

E = 16
D = 768
H = 128
BT = 256
DP = D + 128
NW = 32
N = 4096
CHUNK = N // NW
NB = N // BT
T_TILES = NB + E - 1


def _gate_body(xb, wg, eid, gate, pos_mid, totals, acc):
    b = pl.program_id(0)
    x = xb[...]
    logits = lax.dot_general(x, wg[...], (((1,), (1,)), ((), ())),
                             preferred_element_type=jnp.float32)
    m = jnp.max(logits, axis=1, keepdims=True)
    ex = jnp.exp(logits - m)
    p = ex / jnp.sum(ex, axis=1, keepdims=True)
    top = jnp.max(p, axis=1, keepdims=True)
    gate[...] = top / (top + 1e-6)
    amax = jnp.argmax(p, axis=1).reshape(BT, 1).astype(jnp.int32)
    eid[...] = amax

    ecol = lax.broadcasted_iota(jnp.int32, (BT, E), 1)
    onehot = jnp.where(amax == ecol, 1.0, 0.0)
    ri = lax.broadcasted_iota(jnp.int32, (BT, BT), 0)
    ci = lax.broadcasted_iota(jnp.int32, (BT, BT), 1)
    tri = jnp.where(ri > ci, 1.0, 0.0)
    rank_mat = lax.dot_general(tri, onehot, (((1,), (0,)), ((), ())),
                               preferred_element_type=jnp.float32)
    rank = jnp.sum(rank_mat * onehot, axis=1, keepdims=True)

    @pl.when(b == 0)
    def _():
        acc[...] = jnp.zeros_like(acc)

    a = acc[0:1, 0:E]
    basec = jnp.sum(a * onehot, axis=1, keepdims=True)
    pos_mid[...] = (rank + basec).astype(jnp.int32)
    hist = jnp.sum(onehot, axis=0, keepdims=True)
    acc[0:1, 0:E] = a + hist
    totals[...] = (a + hist).astype(jnp.int32)


def _gating(xf, wg):
    return pl.pallas_call(
        _gate_body,
        grid=(NB,),
        in_specs=[
            pl.BlockSpec((BT, D), lambda b: (b, 0)),
            pl.BlockSpec((E, D), lambda b: (0, 0)),
        ],
        out_specs=[
            pl.BlockSpec((BT, 1), lambda b: (b, 0)),
            pl.BlockSpec((BT, 1), lambda b: (b, 0)),
            pl.BlockSpec((BT, 1), lambda b: (b, 0)),
            pl.BlockSpec((1, E), lambda b: (0, 0)),
        ],
        out_shape=[
            jax.ShapeDtypeStruct((N, 1), jnp.int32),
            jax.ShapeDtypeStruct((N, 1), jnp.float32),
            jax.ShapeDtypeStruct((N, 1), jnp.int32),
            jax.ShapeDtypeStruct((1, E), jnp.int32),
        ],
        scratch_shapes=[pltpu.VMEM((8, 128), jnp.float32)],
    )(xf, wg)


def _dispatch_body(xf_hbm, eid_hbm, gate_hbm, posmid_hbm, totals_hbm,
                   xs_hbm, pos_hbm,
                   eid_v, pm_v, gate_v, tot_v, pos_v, rows_v, sem, sem2):
    wid = lax.axis_index("s") * 2 + lax.axis_index("c")
    base = wid * CHUNK
    xcp = pltpu.async_copy(xf_hbm.at[pl.ds(base, CHUNK)],
                           rows_v.at[:, pl.ds(0, D)], sem2)
    pltpu.sync_copy(eid_hbm.at[pl.ds(base, CHUNK)], eid_v)
    pltpu.sync_copy(posmid_hbm.at[pl.ds(base, CHUNK)], pm_v)
    pltpu.sync_copy(gate_hbm.at[pl.ds(base, CHUNK)], gate_v)
    pltpu.sync_copy(totals_hbm, tot_v)

    lanes = lax.iota(jnp.int32, 16)

    def shift_masked(vec, s):
        idx = jnp.maximum(lanes - s, 0)
        sh = vec.at[idx].get(mode="promise_in_bounds")
        return jnp.where(lanes >= s, sh, 0)

    def bcast(vec, k):
        return vec.at[jnp.full((16,), k, jnp.int32)].get(mode="promise_in_bounds")

    incl = tot_v[...]
    for s in (1, 2, 4, 8):
        incl = incl + shift_masked(incl, s)
    gbase = shift_masked(incl, 1)

    for v in range(CHUNK // 16):
        ev = eid_v[pl.ds(v * 16, 16)]
        basev = gbase.at[ev].get(mode="promise_in_bounds")
        pos_v[pl.ds(v * 16, 16)] = basev + pm_v[pl.ds(v * 16, 16)]
        gv = gate_v[pl.ds(v * 16, 16)]
        for k in range(16):
            rows_v.at[v * 16 + k][pl.ds(D, 16)] = bcast(gv, k)
    pltpu.sync_copy(pos_v, pos_hbm.at[pl.ds(base, CHUNK)])

    xcp.wait()
    pltpu.async_copy(rows_v, xs_hbm.at[pos_v], sem).wait()


def _dispatch(xf, eid, gate, pos_mid, totals):
    mesh = plsc.VectorSubcoreMesh(core_axis_name="c", subcore_axis_name="s")
    return pl.kernel(
        _dispatch_body,
        out_type=(
            jax.ShapeDtypeStruct((N, DP), jnp.float32),
            jax.ShapeDtypeStruct((N,), jnp.int32),
        ),
        mesh=mesh,
        scratch_types=[
            pltpu.VMEM((CHUNK,), jnp.int32),
            pltpu.VMEM((CHUNK,), jnp.int32),
            pltpu.VMEM((CHUNK,), jnp.float32),
            pltpu.VMEM((16,), jnp.int32),
            pltpu.VMEM((CHUNK,), jnp.int32),
            pltpu.VMEM((CHUNK, DP), jnp.float32),
            pltpu.SemaphoreType.DMA,
            pltpu.SemaphoreType.DMA,
        ],
    )(xf, eid, gate, pos_mid, totals)


def _tile_schedule(totals):
    sizes = totals
    off = jnp.cumsum(sizes) - sizes
    end = off + sizes
    first_blk = off // BT
    last_blk = (end - 1) // BT
    tiles = jnp.where(sizes > 0, last_blk - first_blk + 1, 0)
    tstart = jnp.cumsum(tiles) - tiles

    t = jnp.arange(T_TILES, dtype=jnp.int32)[:, None]
    valid = (t >= tstart[None, :]) & (t < (tstart + tiles)[None, :])
    any_valid = valid.any(axis=1)
    ex_id = jnp.sum(jnp.where(valid, jnp.arange(E, dtype=jnp.int32)[None, :], 0), axis=1)
    k = t[:, 0] - jnp.sum(jnp.where(valid, tstart[None, :], 0), axis=1)
    blk = jnp.sum(jnp.where(valid, first_blk[None, :], 0), axis=1) + k
    blk = jnp.where(any_valid, blk, NB - 1)
    lo = jnp.clip(off[ex_id] - blk * BT, 0, BT)
    hi = jnp.clip(end[ex_id] - blk * BT, 0, BT)
    lo = jnp.where(any_valid, lo, 0)
    hi = jnp.where(any_valid, hi, 0)
    return (blk.astype(jnp.int32), ex_id.astype(jnp.int32),
            lo.astype(jnp.int32), hi.astype(jnp.int32))


def _ffn_body(blk, ex, lo, hi, xs_b, w_in_e, b_in_e, w_out_e, b_out_e, out):
    t = pl.program_id(0)
    x = xs_b[:, :D].astype(jnp.bfloat16)
    g = xs_b[:, D:D + 1]
    h = lax.dot_general(x, w_in_e[0], (((1,), (0,)), ((), ())),
                        preferred_element_type=jnp.float32) + b_in_e[0, 0]
    h = h * 0.5 * (1.0 + lax.erf(h * 0.7071067811865476))
    o = lax.dot_general(h.astype(jnp.bfloat16), w_out_e[0],
                        (((1,), (0,)), ((), ())),
                        preferred_element_type=jnp.float32) + b_out_e[0, 0]
    o = o * g
    row = lax.broadcasted_iota(jnp.int32, (BT, 1), 0)
    mask = (row >= lo[t]) & (row < hi[t])
    contrib = jnp.where(mask, o, 0.0)
    first = jnp.logical_or(t == 0, blk[t] != blk[lax.max(t - 1, 0)])

    @pl.when(first)
    def _init():
        out[...] = contrib

    @pl.when(jnp.logical_not(first))
    def _acc():
        out[...] = out[...] + contrib


def _grouped_ffn(xs, w_in, b_in, w_out, b_out, sched):
    grid_spec = pltpu.PrefetchScalarGridSpec(
        num_scalar_prefetch=4,
        grid=(T_TILES,),
        in_specs=[
            pl.BlockSpec((BT, DP), lambda t, blk, ex, lo, hi: (blk[t], 0)),
            pl.BlockSpec((1, D, H), lambda t, blk, ex, lo, hi: (ex[t], 0, 0)),
            pl.BlockSpec((1, 1, H), lambda t, blk, ex, lo, hi: (ex[t], 0, 0)),
            pl.BlockSpec((1, H, D), lambda t, blk, ex, lo, hi: (ex[t], 0, 0)),
            pl.BlockSpec((1, 1, D), lambda t, blk, ex, lo, hi: (ex[t], 0, 0)),
        ],
        out_specs=pl.BlockSpec((BT, D), lambda t, blk, ex, lo, hi: (blk[t], 0)),
    )
    return pl.pallas_call(
        _ffn_body,
        grid_spec=grid_spec,
        out_shape=jax.ShapeDtypeStruct((N, D), jnp.float32),
    )(*sched, xs, w_in.astype(jnp.bfloat16), b_in.reshape(E, 1, H),
      w_out.astype(jnp.bfloat16), b_out.reshape(E, 1, D))


def _combine_body(ys_hbm, pos_hbm, y_hbm, pos_v, rows_v, sem):
    wid = lax.axis_index("s") * 2 + lax.axis_index("c")
    base = wid * CHUNK
    pltpu.sync_copy(pos_hbm.at[pl.ds(base, CHUNK)], pos_v)
    pltpu.async_copy(ys_hbm.at[pos_v], rows_v, sem).wait()
    pltpu.sync_copy(rows_v, y_hbm.at[pl.ds(base, CHUNK)])


def _combine(ys, pos):
    mesh = plsc.VectorSubcoreMesh(core_axis_name="c", subcore_axis_name="s")
    return pl.kernel(
        _combine_body,
        out_type=jax.ShapeDtypeStruct((N + 8, D), jnp.float32),
        mesh=mesh,
        scratch_types=[
            pltpu.VMEM((CHUNK,), jnp.int32),
            pltpu.VMEM((CHUNK, D), jnp.float32),
            pltpu.SemaphoreType.DMA,
        ],
    )(ys, pos)


def kernel(x, task_bh, w_gate, w_in, b_in, w_out, b_out):
    bsz, length, d = x.shape
    xf = x.reshape(-1, d)
    wg = w_gate[task_bh]
    eid, gate, pos_mid, totals = _gating(xf, wg)
    totals = totals.reshape(E)
    xs, pos = _dispatch(xf, eid.reshape(-1), gate.reshape(-1),
                        pos_mid.reshape(-1), totals)
    sched = _tile_schedule(totals)
    ys = _grouped_ffn(xs, w_in, b_in, w_out, b_out, sched)
    y = _combine(ys, pos)[:N]
    return y.reshape(bsz, length, d)


# speedup vs baseline: 1.1486x; 1.1486x over previous
import functools

import jax
import jax.numpy as jnp
from jax import lax
from jax.experimental import pallas as pl
from jax.experimental.pallas import tpu as pltpu
from jax.experimental.pallas import tpu_sc as plsc

E = 16
D = 768
H = 128
BT = 256
DP = D + 128
NW = 32
N = 4096
CHUNK = N // NW
NB = N // BT
T_TILES = NB + E - 1


def _gate_body(tb, xb, wg, eid, gate, pos_mid, totals, acc):
    b = pl.program_id(0)
    x = xb[...]
    logits = lax.dot_general(x, wg[0], (((1,), (1,)), ((), ())),
                             preferred_element_type=jnp.float32)
    m = jnp.max(logits, axis=1, keepdims=True)
    ex = jnp.exp(logits - m)
    p = ex / jnp.sum(ex, axis=1, keepdims=True)
    top = jnp.max(p, axis=1, keepdims=True)
    gate[...] = top / (top + 1e-6)
    amax = jnp.argmax(p, axis=1).reshape(BT, 1).astype(jnp.int32)
    eid[...] = amax

    ecol = lax.broadcasted_iota(jnp.int32, (BT, E), 1)
    onehot = jnp.where(amax == ecol, 1.0, 0.0)
    ri = lax.broadcasted_iota(jnp.int32, (BT, BT), 0)
    ci = lax.broadcasted_iota(jnp.int32, (BT, BT), 1)
    tri = jnp.where(ri > ci, 1.0, 0.0)
    rank_mat = lax.dot_general(tri, onehot, (((1,), (0,)), ((), ())),
                               preferred_element_type=jnp.float32)
    rank = jnp.sum(rank_mat * onehot, axis=1, keepdims=True)

    @pl.when(b == 0)
    def _():
        acc[...] = jnp.zeros_like(acc)

    a = acc[0:1, 0:E]
    basec = jnp.sum(a * onehot, axis=1, keepdims=True)
    pos_mid[...] = (rank + basec).astype(jnp.int32)
    hist = jnp.sum(onehot, axis=0, keepdims=True)
    acc[0:1, 0:E] = a + hist
    totals[...] = (a + hist).astype(jnp.int32)


def _gating(xf, w_gate, tb):
    grid_spec = pltpu.PrefetchScalarGridSpec(
        num_scalar_prefetch=1,
        grid=(NB,),
        in_specs=[
            pl.BlockSpec((BT, D), lambda b, tb: (b, 0)),
            pl.BlockSpec((1, E, D), lambda b, tb: (tb[0], 0, 0)),
        ],
        out_specs=[
            pl.BlockSpec((BT, 1), lambda b, tb: (b, 0)),
            pl.BlockSpec((BT, 1), lambda b, tb: (b, 0)),
            pl.BlockSpec((BT, 1), lambda b, tb: (b, 0)),
            pl.BlockSpec((1, E), lambda b, tb: (0, 0)),
        ],
        scratch_shapes=[pltpu.VMEM((8, 128), jnp.float32)],
    )
    return pl.pallas_call(
        _gate_body,
        grid_spec=grid_spec,
        out_shape=[
            jax.ShapeDtypeStruct((N, 1), jnp.int32),
            jax.ShapeDtypeStruct((N, 1), jnp.float32),
            jax.ShapeDtypeStruct((N, 1), jnp.int32),
            jax.ShapeDtypeStruct((1, E), jnp.int32),
        ],
    )(tb, xf, w_gate)


def _dispatch_body(xf_hbm, eid_hbm, gate_hbm, posmid_hbm, totals_hbm,
                   xs_hbm, pos_hbm,
                   eid_v, pm_v, gate_v, tot_v, pos_v, rows_v, sem, sem2):
    wid = lax.axis_index("s") * 2 + lax.axis_index("c")
    base = wid * CHUNK
    xcp = pltpu.async_copy(xf_hbm.at[pl.ds(base, CHUNK)],
                           rows_v.at[:, pl.ds(0, D)], sem2)
    pltpu.sync_copy(eid_hbm.at[pl.ds(base, CHUNK)], eid_v)
    pltpu.sync_copy(posmid_hbm.at[pl.ds(base, CHUNK)], pm_v)
    pltpu.sync_copy(gate_hbm.at[pl.ds(base, CHUNK)], gate_v)
    pltpu.sync_copy(totals_hbm, tot_v)

    lanes = lax.iota(jnp.int32, 16)

    def shift_masked(vec, s):
        idx = jnp.maximum(lanes - s, 0)
        sh = vec.at[idx].get(mode="promise_in_bounds")
        return jnp.where(lanes >= s, sh, 0)

    def bcast(vec, k):
        return vec.at[jnp.full((16,), k, jnp.int32)].get(mode="promise_in_bounds")

    incl = tot_v[...]
    for s in (1, 2, 4, 8):
        incl = incl + shift_masked(incl, s)
    gbase = shift_masked(incl, 1)

    for v in range(CHUNK // 16):
        ev = eid_v[pl.ds(v * 16, 16)]
        basev = gbase.at[ev].get(mode="promise_in_bounds")
        pos_v[pl.ds(v * 16, 16)] = basev + pm_v[pl.ds(v * 16, 16)]
        gv = gate_v[pl.ds(v * 16, 16)]
        for k in range(16):
            rows_v.at[v * 16 + k][pl.ds(D, 16)] = bcast(gv, k)
    pltpu.sync_copy(pos_v, pos_hbm.at[pl.ds(base, CHUNK)])

    xcp.wait()
    pltpu.async_copy(rows_v, xs_hbm.at[pos_v], sem).wait()


def _dispatch(xf, eid, gate, pos_mid, totals):
    mesh = plsc.VectorSubcoreMesh(core_axis_name="c", subcore_axis_name="s")
    return pl.kernel(
        _dispatch_body,
        out_type=(
            jax.ShapeDtypeStruct((N, DP), jnp.float32),
            jax.ShapeDtypeStruct((N,), jnp.int32),
        ),
        mesh=mesh,
        scratch_types=[
            pltpu.VMEM((CHUNK,), jnp.int32),
            pltpu.VMEM((CHUNK,), jnp.int32),
            pltpu.VMEM((CHUNK,), jnp.float32),
            pltpu.VMEM((16,), jnp.int32),
            pltpu.VMEM((CHUNK,), jnp.int32),
            pltpu.VMEM((CHUNK, DP), jnp.float32),
            pltpu.SemaphoreType.DMA,
            pltpu.SemaphoreType.DMA,
        ],
    )(xf, eid, gate, pos_mid, totals)


def _tile_schedule(totals):
    sizes = totals
    off = jnp.cumsum(sizes) - sizes
    end = off + sizes
    first_blk = off // BT
    last_blk = (end - 1) // BT
    tiles = jnp.where(sizes > 0, last_blk - first_blk + 1, 0)
    tstart = jnp.cumsum(tiles) - tiles

    t = jnp.arange(T_TILES, dtype=jnp.int32)[:, None]
    valid = (t >= tstart[None, :]) & (t < (tstart + tiles)[None, :])
    any_valid = valid.any(axis=1)
    ex_id = jnp.sum(jnp.where(valid, jnp.arange(E, dtype=jnp.int32)[None, :], 0), axis=1)
    k = t[:, 0] - jnp.sum(jnp.where(valid, tstart[None, :], 0), axis=1)
    blk = jnp.sum(jnp.where(valid, first_blk[None, :], 0), axis=1) + k
    blk = jnp.where(any_valid, blk, NB - 1)
    lo = jnp.clip(off[ex_id] - blk * BT, 0, BT)
    hi = jnp.clip(end[ex_id] - blk * BT, 0, BT)
    lo = jnp.where(any_valid, lo, 0)
    hi = jnp.where(any_valid, hi, 0)
    return (blk.astype(jnp.int32), ex_id.astype(jnp.int32),
            lo.astype(jnp.int32), hi.astype(jnp.int32))


def _ffn_body(blk, ex, lo, hi, xs_b, w_in_e, b_in_e, w_out_e, b_out_e, out):
    t = pl.program_id(0)
    x = xs_b[:, :D]
    g = xs_b[:, D:D + 1]
    h = lax.dot_general(x, w_in_e[0], (((1,), (0,)), ((), ())),
                        preferred_element_type=jnp.float32) + b_in_e[0, 0]
    h = h * 0.5 * (1.0 + lax.erf(h * 0.7071067811865476))
    o = lax.dot_general(h, w_out_e[0], (((1,), (0,)), ((), ())),
                        preferred_element_type=jnp.float32) + b_out_e[0, 0]
    o = o * g
    row = lax.broadcasted_iota(jnp.int32, (BT, 1), 0)
    mask = (row >= lo[t]) & (row < hi[t])
    contrib = jnp.where(mask, o, 0.0)
    first = jnp.logical_or(t == 0, blk[t] != blk[lax.max(t - 1, 0)])

    @pl.when(first)
    def _init():
        out[...] = contrib

    @pl.when(jnp.logical_not(first))
    def _acc():
        out[...] = out[...] + contrib


def _grouped_ffn(xs, w_in, b_in, w_out, b_out, sched):
    grid_spec = pltpu.PrefetchScalarGridSpec(
        num_scalar_prefetch=4,
        grid=(T_TILES,),
        in_specs=[
            pl.BlockSpec((BT, DP), lambda t, blk, ex, lo, hi: (blk[t], 0)),
            pl.BlockSpec((1, D, H), lambda t, blk, ex, lo, hi: (ex[t], 0, 0)),
            pl.BlockSpec((1, 1, H), lambda t, blk, ex, lo, hi: (ex[t], 0, 0)),
            pl.BlockSpec((1, H, D), lambda t, blk, ex, lo, hi: (ex[t], 0, 0)),
            pl.BlockSpec((1, 1, D), lambda t, blk, ex, lo, hi: (ex[t], 0, 0)),
        ],
        out_specs=pl.BlockSpec((BT, D), lambda t, blk, ex, lo, hi: (blk[t], 0)),
    )
    return pl.pallas_call(
        _ffn_body,
        grid_spec=grid_spec,
        out_shape=jax.ShapeDtypeStruct((N + BT, D), jnp.float32),
    )(*sched, xs, w_in, b_in.reshape(E, 1, H), w_out, b_out.reshape(E, 1, D))


def _combine_body(ys_hbm, pos_hbm, y_hbm, pos_v, rows_v, sem):
    wid = lax.axis_index("s") * 2 + lax.axis_index("c")
    base = wid * CHUNK
    pltpu.sync_copy(pos_hbm.at[pl.ds(base, CHUNK)], pos_v)
    pltpu.async_copy(ys_hbm.at[pos_v], rows_v, sem).wait()
    pltpu.sync_copy(rows_v, y_hbm.at[pl.ds(base, CHUNK)])


def _combine(ys, pos):
    mesh = plsc.VectorSubcoreMesh(core_axis_name="c", subcore_axis_name="s")
    return pl.kernel(
        _combine_body,
        out_type=jax.ShapeDtypeStruct((N, D), jnp.float32),
        mesh=mesh,
        scratch_types=[
            pltpu.VMEM((CHUNK,), jnp.int32),
            pltpu.VMEM((CHUNK, D), jnp.float32),
            pltpu.SemaphoreType.DMA,
        ],
    )(ys, pos)


def kernel(x, task_bh, w_gate, w_in, b_in, w_out, b_out):
    bsz, length, d = x.shape
    xf = x.reshape(-1, d)
    tb = jnp.asarray(task_bh, jnp.int32).reshape(1)
    eid, gate, pos_mid, totals = _gating(xf, w_gate, tb)
    totals = totals.reshape(E)
    xs, pos = _dispatch(xf, eid.reshape(-1), gate.reshape(-1),
                        pos_mid.reshape(-1), totals)
    sched = _tile_schedule(totals)
    ys = _grouped_ffn(xs, w_in, b_in, w_out, b_out, sched)
    y = _combine(ys, pos)
    return y.reshape(bsz, length, d)


# speedup vs baseline: 1.1505x; 1.0016x over previous
import functools

import jax
import jax.numpy as jnp
from jax import lax
from jax.experimental import pallas as pl
from jax.experimental.pallas import tpu as pltpu
from jax.experimental.pallas import tpu_sc as plsc

E = 16
D = 768
H = 128
BT = 256
DP = D + 128
NW = 32
N = 4096
CHUNK = N // NW
NB = N // BT
T_TILES = NB + E - 1


def _gate_body(tb, xb, wg, eid, gate, pos_mid, totals, acc):
    b = pl.program_id(0)
    x = xb[...]
    logits = lax.dot_general(x, wg[0], (((1,), (1,)), ((), ())),
                             preferred_element_type=jnp.float32)
    m = jnp.max(logits, axis=1, keepdims=True)
    ex = jnp.exp(logits - m)
    p = ex / jnp.sum(ex, axis=1, keepdims=True)
    top = jnp.max(p, axis=1, keepdims=True)
    gate[...] = top / (top + 1e-6)
    amax = jnp.argmax(p, axis=1).reshape(BT, 1).astype(jnp.int32)
    eid[...] = amax

    ecol = lax.broadcasted_iota(jnp.int32, (BT, E), 1)
    onehot = jnp.where(amax == ecol, 1.0, 0.0)
    ri = lax.broadcasted_iota(jnp.int32, (BT, BT), 0)
    ci = lax.broadcasted_iota(jnp.int32, (BT, BT), 1)
    tri = jnp.where(ri > ci, 1.0, 0.0)
    rank_mat = lax.dot_general(tri, onehot, (((1,), (0,)), ((), ())),
                               preferred_element_type=jnp.float32)
    rank = jnp.sum(rank_mat * onehot, axis=1, keepdims=True)

    @pl.when(b == 0)
    def _():
        acc[...] = jnp.zeros_like(acc)

    a = acc[0:1, 0:E]
    basec = jnp.sum(a * onehot, axis=1, keepdims=True)
    pos_mid[...] = (rank + basec).astype(jnp.int32)
    hist = jnp.sum(onehot, axis=0, keepdims=True)
    acc[0:1, 0:E] = a + hist
    totals[...] = (a + hist).astype(jnp.int32)


def _tile_schedule_jnp(sizes):
    off = jnp.cumsum(sizes) - sizes
    end = off + sizes
    first_blk = off // BT
    last_blk = (end - 1) // BT
    tiles = jnp.where(sizes > 0, last_blk - first_blk + 1, 0)
    tstart = jnp.cumsum(tiles) - tiles
    t = jnp.arange(T_TILES, dtype=jnp.int32)[:, None]
    valid = (t >= tstart[None, :]) & (t < (tstart + tiles)[None, :])
    any_valid = valid.any(axis=1)
    ex_id = jnp.sum(jnp.where(valid, jnp.arange(E, dtype=jnp.int32)[None, :], 0), axis=1)
    k = t[:, 0] - jnp.sum(jnp.where(valid, tstart[None, :], 0), axis=1)
    blk = jnp.sum(jnp.where(valid, first_blk[None, :], 0), axis=1) + k
    blk = jnp.where(any_valid, blk, NB - 1)
    lo = jnp.clip(off[ex_id] - blk * BT, 0, BT)
    hi = jnp.clip(end[ex_id] - blk * BT, 0, BT)
    lo = jnp.where(any_valid, lo, 0)
    hi = jnp.where(any_valid, hi, 0)
    return (blk.astype(jnp.int32), ex_id.astype(jnp.int32),
            lo.astype(jnp.int32), hi.astype(jnp.int32))


def _gating(xf, w_gate, tb):
    grid_spec = pltpu.PrefetchScalarGridSpec(
        num_scalar_prefetch=1,
        grid=(NB,),
        in_specs=[
            pl.BlockSpec((BT, D), lambda b, tb: (b, 0)),
            pl.BlockSpec((1, E, D), lambda b, tb: (tb[0], 0, 0)),
        ],
        out_specs=[
            pl.BlockSpec((BT, 1), lambda b, tb: (b, 0)),
            pl.BlockSpec((BT, 1), lambda b, tb: (b, 0)),
            pl.BlockSpec((BT, 1), lambda b, tb: (b, 0)),
            pl.BlockSpec((1, E), lambda b, tb: (0, 0)),
        ],
        scratch_shapes=[pltpu.VMEM((8, 128), jnp.float32)],
    )
    return pl.pallas_call(
        _gate_body,
        grid_spec=grid_spec,
        out_shape=[
            jax.ShapeDtypeStruct((N, 1), jnp.int32),
            jax.ShapeDtypeStruct((N, 1), jnp.float32),
            jax.ShapeDtypeStruct((N, 1), jnp.int32),
            jax.ShapeDtypeStruct((1, E), jnp.int32),
        ],
    )(tb, xf, w_gate)


def _dispatch_body(xf_hbm, eid_hbm, gate_hbm, posmid_hbm, totals_hbm,
                   xs_hbm, pos_hbm,
                   eid_v, pm_v, gate_v, tot_v, pos_v, rows_v, sem, sem2):
    wid = lax.axis_index("s") * 2 + lax.axis_index("c")
    base = wid * CHUNK
    xcp = pltpu.async_copy(xf_hbm.at[pl.ds(base, CHUNK)],
                           rows_v.at[:, pl.ds(0, D)], sem2)
    pltpu.sync_copy(eid_hbm.at[pl.ds(base, CHUNK)], eid_v)
    pltpu.sync_copy(posmid_hbm.at[pl.ds(base, CHUNK)], pm_v)
    pltpu.sync_copy(gate_hbm.at[pl.ds(base, CHUNK)], gate_v)
    pltpu.sync_copy(totals_hbm, tot_v)

    lanes = lax.iota(jnp.int32, 16)

    def shift_masked(vec, s):
        idx = jnp.maximum(lanes - s, 0)
        sh = vec.at[idx].get(mode="promise_in_bounds")
        return jnp.where(lanes >= s, sh, 0)

    def bcast(vec, k):
        return vec.at[jnp.full((16,), k, jnp.int32)].get(mode="promise_in_bounds")

    incl = tot_v[...]
    for s in (1, 2, 4, 8):
        incl = incl + shift_masked(incl, s)
    gbase = shift_masked(incl, 1)

    for v in range(CHUNK // 16):
        ev = eid_v[pl.ds(v * 16, 16)]
        basev = gbase.at[ev].get(mode="promise_in_bounds")
        pos_v[pl.ds(v * 16, 16)] = basev + pm_v[pl.ds(v * 16, 16)]
        gv = gate_v[pl.ds(v * 16, 16)]
        for k in range(16):
            rows_v.at[v * 16 + k][pl.ds(D, 16)] = bcast(gv, k)
    pltpu.sync_copy(pos_v, pos_hbm.at[pl.ds(base, CHUNK)])

    xcp.wait()
    pltpu.async_copy(rows_v, xs_hbm.at[pos_v], sem).wait()


def _dispatch(xf, eid, gate, pos_mid, totals):
    mesh = plsc.VectorSubcoreMesh(core_axis_name="c", subcore_axis_name="s")
    return pl.kernel(
        _dispatch_body,
        out_type=(
            jax.ShapeDtypeStruct((N, DP), jnp.float32),
            jax.ShapeDtypeStruct((N,), jnp.int32),
        ),
        mesh=mesh,
        scratch_types=[
            pltpu.VMEM((CHUNK,), jnp.int32),
            pltpu.VMEM((CHUNK,), jnp.int32),
            pltpu.VMEM((CHUNK,), jnp.float32),
            pltpu.VMEM((16,), jnp.int32),
            pltpu.VMEM((CHUNK,), jnp.int32),
            pltpu.VMEM((CHUNK, DP), jnp.float32),
            pltpu.SemaphoreType.DMA,
            pltpu.SemaphoreType.DMA,
        ],
    )(xf, eid, gate, pos_mid, totals)


def _ffn_body(blk, ex, lo, hi, xs_b, w_in_e, b_in_e, w_out_e, b_out_e, out):
    t = pl.program_id(0)
    x = xs_b[:, :D]
    g = xs_b[:, D:D + 1]
    h = lax.dot_general(x, w_in_e[0], (((1,), (0,)), ((), ())),
                        preferred_element_type=jnp.float32) + b_in_e[0, 0]
    h = h * 0.5 * (1.0 + lax.erf(h * 0.7071067811865476))
    o = lax.dot_general(h, w_out_e[0], (((1,), (0,)), ((), ())),
                        preferred_element_type=jnp.float32) + b_out_e[0, 0]
    o = o * g
    row = lax.broadcasted_iota(jnp.int32, (BT, 1), 0)
    mask = (row >= lo[t]) & (row < hi[t])
    contrib = jnp.where(mask, o, 0.0)
    first = jnp.logical_or(t == 0, blk[t] != blk[lax.max(t - 1, 0)])

    @pl.when(first)
    def _init():
        out[...] = contrib

    @pl.when(jnp.logical_not(first))
    def _acc():
        out[...] = out[...] + contrib


def _grouped_ffn(xs, w_in, b_in, w_out, b_out, sched):
    grid_spec = pltpu.PrefetchScalarGridSpec(
        num_scalar_prefetch=4,
        grid=(T_TILES,),
        in_specs=[
            pl.BlockSpec((BT, DP), lambda t, blk, ex, lo, hi: (blk[t], 0)),
            pl.BlockSpec((1, D, H), lambda t, blk, ex, lo, hi: (ex[t], 0, 0)),
            pl.BlockSpec((1, 1, H), lambda t, blk, ex, lo, hi: (ex[t], 0, 0)),
            pl.BlockSpec((1, H, D), lambda t, blk, ex, lo, hi: (ex[t], 0, 0)),
            pl.BlockSpec((1, 1, D), lambda t, blk, ex, lo, hi: (ex[t], 0, 0)),
        ],
        out_specs=pl.BlockSpec((BT, D), lambda t, blk, ex, lo, hi: (blk[t], 0)),
    )
    return pl.pallas_call(
        _ffn_body,
        grid_spec=grid_spec,
        out_shape=jax.ShapeDtypeStruct((N + BT, D), jnp.float32),
    )(*sched,
      xs, w_in, b_in.reshape(E, 1, H), w_out, b_out.reshape(E, 1, D))


def _combine_body(ys_hbm, pos_hbm, y_hbm, pos_v, rows_v, sem):
    wid = lax.axis_index("s") * 2 + lax.axis_index("c")
    base = wid * CHUNK
    pltpu.sync_copy(pos_hbm.at[pl.ds(base, CHUNK)], pos_v)
    pltpu.async_copy(ys_hbm.at[pos_v], rows_v, sem).wait()
    pltpu.sync_copy(rows_v, y_hbm.at[pl.ds(base, CHUNK)])


def _combine(ys, pos):
    mesh = plsc.VectorSubcoreMesh(core_axis_name="c", subcore_axis_name="s")
    return pl.kernel(
        _combine_body,
        out_type=jax.ShapeDtypeStruct((N, D), jnp.float32),
        mesh=mesh,
        scratch_types=[
            pltpu.VMEM((CHUNK,), jnp.int32),
            pltpu.VMEM((CHUNK, D), jnp.float32),
            pltpu.SemaphoreType.DMA,
        ],
    )(ys, pos)


def kernel(x, task_bh, w_gate, w_in, b_in, w_out, b_out):
    bsz, length, d = x.shape
    xf = x.reshape(-1, d)
    tb = jnp.asarray(task_bh, jnp.int32).reshape(1)
    eid, gate, pos_mid, totals = _gating(xf, w_gate, tb)
    totals = totals.reshape(E)
    xs, pos = _dispatch(xf, eid.reshape(-1), gate.reshape(-1),
                        pos_mid.reshape(-1), totals)
    sched = _tile_schedule_jnp(totals)
    ys = _grouped_ffn(xs, w_in, b_in, w_out, b_out, sched)
    y = _combine(ys, pos)
    return y.reshape(bsz, length, d)
